# baseline (speedup 1.0000x reference)
import jax
import jax.numpy as jnp
from jax import lax
from jax.experimental import pallas as pl
from jax.experimental.pallas import tpu as pltpu
from jax.experimental.pallas import tpu_sc as plsc

BATCH = 1024
INPUT_DIM = 100000
NUM_RULES = 32768
CHUNK = 4096
LANES = 16
NUM_CORES = 2
NUM_SUBCORES = 16
NUM_WORKERS = NUM_CORES * NUM_SUBCORES
ROWS_PER_W = BATCH // NUM_WORKERS


def _sc_body(facts_hbm, idx1_hbm, idx2_hbm, w_hbm, out_hbm,
             row_v, i1_v, i2_v, w_v, out_v):
    wid = lax.axis_index("s") * NUM_CORES + lax.axis_index("c")

    def row_body(r, carry):
        row = wid * ROWS_PER_W + r
        pltpu.sync_copy(facts_hbm.at[row], row_v)
        for c in range(NUM_RULES // CHUNK):
            base_r = c * CHUNK
            pltpu.sync_copy(idx1_hbm.at[pl.ds(base_r, CHUNK)], i1_v)
            pltpu.sync_copy(idx2_hbm.at[pl.ds(base_r, CHUNK)], i2_v)
            pltpu.sync_copy(w_hbm.at[pl.ds(base_r, CHUNK)], w_v)

            def j_body(j, jcarry):
                b = j * LANES
                i1 = i1_v[pl.ds(b, LANES)]
                i2 = i2_v[pl.ds(b, LANES)]
                w = w_v[pl.ds(b, LANES)]
                s = 1.0 / (1.0 + jnp.exp(-w))
                f1 = plsc.load_gather(row_v, [i1])
                f2 = plsc.load_gather(row_v, [i2])
                out_v[pl.ds(b, LANES)] = s * (f1 * f2)
                return jcarry

            lax.fori_loop(0, CHUNK // LANES, j_body, 0)
            pltpu.sync_copy(out_v, out_hbm.at[row, pl.ds(base_r, CHUNK)])
        return carry

    lax.fori_loop(0, ROWS_PER_W, row_body, 0)


def kernel(facts, idx, weight):
    idx32 = idx.astype(jnp.int32)
    idx1 = idx32[:, 0]
    idx2 = idx32[:, 1]
    mesh = plsc.VectorSubcoreMesh(core_axis_name="c", subcore_axis_name="s")
    f = pl.kernel(
        _sc_body,
        out_type=jax.ShapeDtypeStruct((BATCH, NUM_RULES), jnp.float32),
        mesh=mesh,
        compiler_params=pltpu.CompilerParams(needs_layout_passes=False),
        scratch_types=[
            pltpu.VMEM((INPUT_DIM,), jnp.float32),
            pltpu.VMEM((CHUNK,), jnp.int32),
            pltpu.VMEM((CHUNK,), jnp.int32),
            pltpu.VMEM((CHUNK,), jnp.float32),
            pltpu.VMEM((CHUNK,), jnp.float32),
        ],
    )
    return f(facts, idx1, idx2, weight)


# speedup vs baseline: 1.6234x; 1.6234x over previous
import jax
import jax.numpy as jnp
from jax import lax
from jax.experimental import pallas as pl
from jax.experimental.pallas import tpu as pltpu
from jax.experimental.pallas import tpu_sc as plsc

BATCH = 1024
INPUT_DIM = 100000
NUM_RULES = 32768
CHUNK = 2048
NCHUNK = NUM_RULES // CHUNK
LANES = 16
NUM_CORES = 2
NUM_SUBCORES = 16
NUM_WORKERS = NUM_CORES * NUM_SUBCORES
ROWS_PER_W = BATCH // NUM_WORKERS


def _sigmoid16(w):
    return 1.0 / (1.0 + jnp.exp(-w))


def _sc_body(facts_hbm, idx1_hbm, idx2_hbm, w_hbm, out_hbm,
             row_v, scale_v, i1_v, i2_v, o_v, sem_i, sem_o):
    wid = lax.axis_index("s") * NUM_CORES + lax.axis_index("c")

    for c in range(NCHUNK):
        pltpu.sync_copy(w_hbm.at[pl.ds(c * CHUNK, CHUNK)], o_v[0])

        def s_body(k, carry, c=c):
            b = k * 32
            s0 = _sigmoid16(o_v[0][pl.ds(b, LANES)])
            s1 = _sigmoid16(o_v[0][pl.ds(b + LANES, LANES)])
            scale_v[pl.ds(c * CHUNK + b, 32)] = plsc.pack(
                s0, s1, format=plsc.PackFormat.INTERLEAVED)
            return carry

        lax.fori_loop(0, CHUNK // 32, s_body, 0)

    pltpu.make_async_copy(idx1_hbm.at[pl.ds(0, CHUNK)], i1_v[0], sem_i).start()
    pltpu.make_async_copy(idx2_hbm.at[pl.ds(0, CHUNK)], i2_v[0], sem_i).start()

    def row_body(r, carry):
        row = wid * ROWS_PER_W + r
        pltpu.sync_copy(facts_hbm.at[row], row_v)
        for c in range(NCHUNK):
            slot = c % 2
            nxt = (c + 1) % NCHUNK
            pltpu.make_async_copy(
                idx1_hbm.at[pl.ds(c * CHUNK, CHUNK)], i1_v[slot], sem_i).wait()
            pltpu.make_async_copy(
                idx2_hbm.at[pl.ds(c * CHUNK, CHUNK)], i2_v[slot], sem_i).wait()
            pltpu.make_async_copy(
                idx1_hbm.at[pl.ds(nxt * CHUNK, CHUNK)],
                i1_v[1 - slot], sem_i).start()
            pltpu.make_async_copy(
                idx2_hbm.at[pl.ds(nxt * CHUNK, CHUNK)],
                i2_v[1 - slot], sem_i).start()
            if c >= 2:
                pltpu.make_async_copy(
                    o_v[slot], out_hbm.at[0, pl.ds(0, CHUNK)],
                    sem_o[slot]).wait()
            else:
                @pl.when(r > 0)
                def _():
                    pltpu.make_async_copy(
                        o_v[slot], out_hbm.at[0, pl.ds(0, CHUNK)],
                        sem_o[slot]).wait()

            def j_body(j, jcarry, c=c, slot=slot):
                b = j * 32
                i1a = i1_v[slot][pl.ds(b, LANES)]
                i1b = i1_v[slot][pl.ds(b + LANES, LANES)]
                i2a = i2_v[slot][pl.ds(b, LANES)]
                i2b = i2_v[slot][pl.ds(b + LANES, LANES)]
                s0, s1 = plsc.unpack(
                    scale_v[pl.ds(c * CHUNK + b, 32)],
                    format=plsc.PackFormat.INTERLEAVED)
                f1a = plsc.load_gather(row_v, [i1a])
                f2a = plsc.load_gather(row_v, [i2a])
                f1b = plsc.load_gather(row_v, [i1b])
                f2b = plsc.load_gather(row_v, [i2b])
                o_v[slot][pl.ds(b, LANES)] = s0 * (f1a * f2a)
                o_v[slot][pl.ds(b + LANES, LANES)] = s1 * (f1b * f2b)
                return jcarry

            lax.fori_loop(0, CHUNK // 32, j_body, 0)
            pltpu.make_async_copy(
                o_v[slot], out_hbm.at[row, pl.ds(c * CHUNK, CHUNK)],
                sem_o[slot]).start()
        return carry

    lax.fori_loop(0, ROWS_PER_W, row_body, 0)

    pltpu.make_async_copy(idx1_hbm.at[pl.ds(0, CHUNK)], i1_v[0], sem_i).wait()
    pltpu.make_async_copy(idx2_hbm.at[pl.ds(0, CHUNK)], i2_v[0], sem_i).wait()
    for slot in range(2):
        pltpu.make_async_copy(
            o_v[slot], out_hbm.at[0, pl.ds(0, CHUNK)], sem_o[slot]).wait()


def kernel(facts, idx, weight):
    idx32 = idx.astype(jnp.int32)
    idx1 = idx32[:, 0]
    idx2 = idx32[:, 1]
    mesh = plsc.VectorSubcoreMesh(core_axis_name="c", subcore_axis_name="s")
    f = pl.kernel(
        _sc_body,
        out_type=jax.ShapeDtypeStruct((BATCH, NUM_RULES), jnp.float32),
        mesh=mesh,
        compiler_params=pltpu.CompilerParams(needs_layout_passes=False),
        scratch_types=[
            pltpu.VMEM((INPUT_DIM,), jnp.float32),
            pltpu.VMEM((NUM_RULES,), jnp.bfloat16),
            [pltpu.VMEM((CHUNK,), jnp.int32)] * 2,
            [pltpu.VMEM((CHUNK,), jnp.int32)] * 2,
            [pltpu.VMEM((CHUNK,), jnp.float32)] * 2,
            pltpu.SemaphoreType.DMA,
            [pltpu.SemaphoreType.DMA] * 2,
        ],
    )
    return f(facts, idx1, idx2, weight)


# speedup vs baseline: 1.6982x; 1.0461x over previous
import jax
import jax.numpy as jnp
from jax import lax
from jax.experimental import pallas as pl
from jax.experimental.pallas import tpu as pltpu
from jax.experimental.pallas import tpu_sc as plsc

BATCH = 1024
INPUT_DIM = 100000
NUM_RULES = 32768
CHUNK = 2048
NCHUNK = NUM_RULES // CHUNK
LANES = 16
NUM_CORES = 2
NUM_SUBCORES = 16
NUM_WORKERS = NUM_CORES * NUM_SUBCORES
ROWS_PER_W = BATCH // NUM_WORKERS


def _sigmoid16(w):
    return 1.0 / (1.0 + jnp.exp(-w))


def _sc_body(facts_hbm, idx1_hbm, idx2_hbm, w_hbm, out_hbm,
             row_v, scale_v, i1_v, i2_v, o_v, sem_i, sem_o):
    wid = lax.axis_index("s") * NUM_CORES + lax.axis_index("c")

    for c in range(NCHUNK):
        pltpu.sync_copy(w_hbm.at[pl.ds(c * CHUNK, CHUNK)], o_v[0])

        @plsc.parallel_loop(0, CHUNK // 32, unroll=4)
        def s_body(k, c=c):
            b = k * 32
            s0 = _sigmoid16(o_v[0][pl.ds(b, LANES)])
            s1 = _sigmoid16(o_v[0][pl.ds(b + LANES, LANES)])
            scale_v[pl.ds(c * CHUNK + b, 32)] = plsc.pack(
                s0, s1, format=plsc.PackFormat.INTERLEAVED)

    pltpu.make_async_copy(idx1_hbm.at[pl.ds(0, CHUNK)], i1_v[0], sem_i).start()
    pltpu.make_async_copy(idx2_hbm.at[pl.ds(0, CHUNK)], i2_v[0], sem_i).start()

    def row_body(r, carry):
        row = wid * ROWS_PER_W + r
        pltpu.sync_copy(facts_hbm.at[row], row_v)
        for c in range(NCHUNK):
            slot = c % 2
            nxt = (c + 1) % NCHUNK
            pltpu.make_async_copy(
                idx1_hbm.at[pl.ds(c * CHUNK, CHUNK)], i1_v[slot], sem_i).wait()
            pltpu.make_async_copy(
                idx2_hbm.at[pl.ds(c * CHUNK, CHUNK)], i2_v[slot], sem_i).wait()
            pltpu.make_async_copy(
                idx1_hbm.at[pl.ds(nxt * CHUNK, CHUNK)],
                i1_v[1 - slot], sem_i).start()
            pltpu.make_async_copy(
                idx2_hbm.at[pl.ds(nxt * CHUNK, CHUNK)],
                i2_v[1 - slot], sem_i).start()
            if c >= 2:
                pltpu.make_async_copy(
                    o_v[slot], out_hbm.at[0, pl.ds(0, CHUNK)],
                    sem_o[slot]).wait()
            else:
                @pl.when(r > 0)
                def _():
                    pltpu.make_async_copy(
                        o_v[slot], out_hbm.at[0, pl.ds(0, CHUNK)],
                        sem_o[slot]).wait()

            @plsc.parallel_loop(0, CHUNK // 32, unroll=4)
            def j_body(j, c=c, slot=slot):
                b = j * 32
                i1a = i1_v[slot][pl.ds(b, LANES)]
                i1b = i1_v[slot][pl.ds(b + LANES, LANES)]
                i2a = i2_v[slot][pl.ds(b, LANES)]
                i2b = i2_v[slot][pl.ds(b + LANES, LANES)]
                s0, s1 = plsc.unpack(
                    scale_v[pl.ds(c * CHUNK + b, 32)],
                    format=plsc.PackFormat.INTERLEAVED)
                f1a = plsc.load_gather(row_v, [i1a])
                f2a = plsc.load_gather(row_v, [i2a])
                f1b = plsc.load_gather(row_v, [i1b])
                f2b = plsc.load_gather(row_v, [i2b])
                o_v[slot][pl.ds(b, LANES)] = s0 * (f1a * f2a)
                o_v[slot][pl.ds(b + LANES, LANES)] = s1 * (f1b * f2b)

            pltpu.make_async_copy(
                o_v[slot], out_hbm.at[row, pl.ds(c * CHUNK, CHUNK)],
                sem_o[slot]).start()
        return carry

    lax.fori_loop(0, ROWS_PER_W, row_body, 0)

    pltpu.make_async_copy(idx1_hbm.at[pl.ds(0, CHUNK)], i1_v[0], sem_i).wait()
    pltpu.make_async_copy(idx2_hbm.at[pl.ds(0, CHUNK)], i2_v[0], sem_i).wait()
    for slot in range(2):
        pltpu.make_async_copy(
            o_v[slot], out_hbm.at[0, pl.ds(0, CHUNK)], sem_o[slot]).wait()


def kernel(facts, idx, weight):
    idx32 = idx.astype(jnp.int32)
    idx1 = idx32[:, 0]
    idx2 = idx32[:, 1]
    mesh = plsc.VectorSubcoreMesh(core_axis_name="c", subcore_axis_name="s")
    f = pl.kernel(
        _sc_body,
        out_type=jax.ShapeDtypeStruct((BATCH, NUM_RULES), jnp.float32),
        mesh=mesh,
        compiler_params=pltpu.CompilerParams(needs_layout_passes=False),
        scratch_types=[
            pltpu.VMEM((INPUT_DIM,), jnp.float32),
            pltpu.VMEM((NUM_RULES,), jnp.bfloat16),
            [pltpu.VMEM((CHUNK,), jnp.int32)] * 2,
            [pltpu.VMEM((CHUNK,), jnp.int32)] * 2,
            [pltpu.VMEM((CHUNK,), jnp.float32)] * 2,
            pltpu.SemaphoreType.DMA,
            [pltpu.SemaphoreType.DMA] * 2,
        ],
    )
    return f(facts, idx1, idx2, weight)


# speedup vs baseline: 2.1076x; 1.2410x over previous
import jax
import jax.numpy as jnp
from jax import lax
from jax.experimental import pallas as pl
from jax.experimental.pallas import tpu as pltpu
from jax.experimental.pallas import tpu_sc as plsc

BATCH = 1024
INPUT_DIM = 100000
NUM_RULES = 32768
CHUNK = 1024
NCHUNK = NUM_RULES // CHUNK
LANES = 16
NUM_CORES = 2
NUM_SUBCORES = 16
NUM_WORKERS = NUM_CORES * NUM_SUBCORES
ROWS_PER_W = BATCH // NUM_WORKERS


def _sigmoid16(w):
    return 1.0 / (1.0 + jnp.exp(-w))


def _sc_body(facts_hbm, idx1_hbm, idx2_hbm, w_hbm, out_hbm,
             row_v, scale_v, i1_v, i2_v, o_v, i1s_v, i2s_v, sem_i, sem_o):
    sid = lax.axis_index("s")
    wid = sid * NUM_CORES + lax.axis_index("c")

    @pl.when(sid == 0)
    def _():
        pltpu.sync_copy(idx1_hbm, i1s_v)
        pltpu.sync_copy(idx2_hbm, i2s_v)

    for c in range(NCHUNK):
        pltpu.sync_copy(w_hbm.at[pl.ds(c * CHUNK, CHUNK)], o_v[0])

        @plsc.parallel_loop(0, CHUNK // 32, unroll=4)
        def s_body(k, c=c):
            b = k * 32
            s0 = _sigmoid16(o_v[0][pl.ds(b, LANES)])
            s1 = _sigmoid16(o_v[0][pl.ds(b + LANES, LANES)])
            scale_v[pl.ds(c * CHUNK + b, 32)] = plsc.pack(
                s0, s1, format=plsc.PackFormat.INTERLEAVED)

    plsc.subcore_barrier()
    pltpu.make_async_copy(i1s_v.at[pl.ds(0, CHUNK)], i1_v[0], sem_i).start()
    pltpu.make_async_copy(i2s_v.at[pl.ds(0, CHUNK)], i2_v[0], sem_i).start()

    def row_body(r, carry):
        row = wid * ROWS_PER_W + r
        pltpu.sync_copy(facts_hbm.at[row], row_v)
        for c in range(NCHUNK):
            slot = c % 2
            nxt = (c + 1) % NCHUNK
            pltpu.make_async_copy(
                i1s_v.at[pl.ds(c * CHUNK, CHUNK)], i1_v[slot], sem_i).wait()
            pltpu.make_async_copy(
                i2s_v.at[pl.ds(c * CHUNK, CHUNK)], i2_v[slot], sem_i).wait()
            pltpu.make_async_copy(
                i1s_v.at[pl.ds(nxt * CHUNK, CHUNK)],
                i1_v[1 - slot], sem_i).start()
            pltpu.make_async_copy(
                i2s_v.at[pl.ds(nxt * CHUNK, CHUNK)],
                i2_v[1 - slot], sem_i).start()
            if c >= 2:
                pltpu.make_async_copy(
                    o_v[slot], out_hbm.at[0, pl.ds(0, CHUNK)],
                    sem_o[slot]).wait()
            else:
                @pl.when(r > 0)
                def _():
                    pltpu.make_async_copy(
                        o_v[slot], out_hbm.at[0, pl.ds(0, CHUNK)],
                        sem_o[slot]).wait()

            @plsc.parallel_loop(0, CHUNK // 32, unroll=4)
            def j_body(j, c=c, slot=slot):
                b = j * 32
                i1a = i1_v[slot][pl.ds(b, LANES)]
                i1b = i1_v[slot][pl.ds(b + LANES, LANES)]
                i2a = i2_v[slot][pl.ds(b, LANES)]
                i2b = i2_v[slot][pl.ds(b + LANES, LANES)]
                s0, s1 = plsc.unpack(
                    scale_v[pl.ds(c * CHUNK + b, 32)],
                    format=plsc.PackFormat.INTERLEAVED)
                f1a = plsc.load_gather(row_v, [i1a])
                f2a = plsc.load_gather(row_v, [i2a])
                f1b = plsc.load_gather(row_v, [i1b])
                f2b = plsc.load_gather(row_v, [i2b])
                o_v[slot][pl.ds(b, LANES)] = s0 * (f1a * f2a)
                o_v[slot][pl.ds(b + LANES, LANES)] = s1 * (f1b * f2b)

            pltpu.make_async_copy(
                o_v[slot], out_hbm.at[row, pl.ds(c * CHUNK, CHUNK)],
                sem_o[slot]).start()
        return carry

    lax.fori_loop(0, ROWS_PER_W, row_body, 0)

    pltpu.make_async_copy(i1s_v.at[pl.ds(0, CHUNK)], i1_v[0], sem_i).wait()
    pltpu.make_async_copy(i2s_v.at[pl.ds(0, CHUNK)], i2_v[0], sem_i).wait()
    for slot in range(2):
        pltpu.make_async_copy(
            o_v[slot], out_hbm.at[0, pl.ds(0, CHUNK)], sem_o[slot]).wait()


def kernel(facts, idx, weight):
    idx32 = idx.astype(jnp.int32)
    idx1 = idx32[:, 0]
    idx2 = idx32[:, 1]
    mesh = plsc.VectorSubcoreMesh(core_axis_name="c", subcore_axis_name="s")
    f = pl.kernel(
        _sc_body,
        out_type=jax.ShapeDtypeStruct((BATCH, NUM_RULES), jnp.float32),
        mesh=mesh,
        compiler_params=pltpu.CompilerParams(needs_layout_passes=False),
        scratch_types=[
            pltpu.VMEM((INPUT_DIM,), jnp.float32),
            pltpu.VMEM((NUM_RULES,), jnp.bfloat16),
            [pltpu.VMEM((CHUNK,), jnp.int32)] * 2,
            [pltpu.VMEM((CHUNK,), jnp.int32)] * 2,
            [pltpu.VMEM((CHUNK,), jnp.float32)] * 2,
            pltpu.VMEM_SHARED((NUM_RULES,), jnp.int32),
            pltpu.VMEM_SHARED((NUM_RULES,), jnp.int32),
            pltpu.SemaphoreType.DMA,
            [pltpu.SemaphoreType.DMA] * 2,
        ],
    )
    return f(facts, idx1, idx2, weight)


# speedup vs baseline: 2.1105x; 1.0014x over previous
import jax
import jax.numpy as jnp
from jax import lax
from jax.experimental import pallas as pl
from jax.experimental.pallas import tpu as pltpu
from jax.experimental.pallas import tpu_sc as plsc

BATCH = 1024
INPUT_DIM = 100000
NUM_RULES = 32768
CHUNK = 1024
NCHUNK = NUM_RULES // CHUNK
LANES = 16
NUM_CORES = 2
NUM_SUBCORES = 16
NUM_WORKERS = NUM_CORES * NUM_SUBCORES
ROWS_PER_W = BATCH // NUM_WORKERS


def _sigmoid16(w):
    return 1.0 / (1.0 + jnp.exp(-w))


def _sc_body(facts_hbm, idx1_hbm, idx2_hbm, w_hbm, out_hbm,
             row_v, scale_v, i1_v, i2_v, o_v, i1s_v, i2s_v, sem_i, sem_o):
    sid = lax.axis_index("s")
    wid = sid * NUM_CORES + lax.axis_index("c")

    @pl.when(sid == 0)
    def _():
        pltpu.sync_copy(idx1_hbm, i1s_v)
        pltpu.sync_copy(idx2_hbm, i2s_v)

    for c in range(NCHUNK):
        pltpu.sync_copy(w_hbm.at[pl.ds(c * CHUNK, CHUNK)], o_v[0])

        @plsc.parallel_loop(0, CHUNK // 32, unroll=4)
        def s_body(k, c=c):
            b = k * 32
            s0 = _sigmoid16(o_v[0][pl.ds(b, LANES)])
            s1 = _sigmoid16(o_v[0][pl.ds(b + LANES, LANES)])
            scale_v[pl.ds(c * CHUNK + b, 32)] = plsc.pack(
                s0, s1, format=plsc.PackFormat.INTERLEAVED)

    plsc.subcore_barrier()
    pltpu.make_async_copy(i1s_v.at[pl.ds(0, CHUNK)], i1_v[0], sem_i).start()
    pltpu.make_async_copy(i2s_v.at[pl.ds(0, CHUNK)], i2_v[0], sem_i).start()

    def row_body(r, carry):
        row = wid * ROWS_PER_W + r
        pltpu.sync_copy(facts_hbm.at[row], row_v)
        for c in range(NCHUNK):
            slot = c % 2
            nxt = (c + 1) % NCHUNK
            pltpu.make_async_copy(
                i1s_v.at[pl.ds(c * CHUNK, CHUNK)], i1_v[slot], sem_i).wait()
            pltpu.make_async_copy(
                i2s_v.at[pl.ds(c * CHUNK, CHUNK)], i2_v[slot], sem_i).wait()
            pltpu.make_async_copy(
                i1s_v.at[pl.ds(nxt * CHUNK, CHUNK)],
                i1_v[1 - slot], sem_i).start()
            pltpu.make_async_copy(
                i2s_v.at[pl.ds(nxt * CHUNK, CHUNK)],
                i2_v[1 - slot], sem_i).start()
            if c >= 2:
                pltpu.make_async_copy(
                    o_v[slot], out_hbm.at[0, pl.ds(0, CHUNK)],
                    sem_o[slot]).wait()
            else:
                @pl.when(r > 0)
                def _():
                    pltpu.make_async_copy(
                        o_v[slot], out_hbm.at[0, pl.ds(0, CHUNK)],
                        sem_o[slot]).wait()

            @plsc.parallel_loop(0, CHUNK // 32, unroll=4)
            def j_body(j, c=c, slot=slot):
                b = j * 32
                i1a = i1_v[slot][pl.ds(b, LANES)]
                i1b = i1_v[slot][pl.ds(b + LANES, LANES)]
                i2a = i2_v[slot][pl.ds(b, LANES)]
                i2b = i2_v[slot][pl.ds(b + LANES, LANES)]
                s0, s1 = plsc.unpack(
                    scale_v[pl.ds(c * CHUNK + b, 32)],
                    format=plsc.PackFormat.INTERLEAVED)
                f1a = plsc.load_gather(row_v, [i1a])
                f2a = plsc.load_gather(row_v, [i2a])
                f1b = plsc.load_gather(row_v, [i1b])
                f2b = plsc.load_gather(row_v, [i2b])
                o_v[slot][pl.ds(b, LANES)] = s0 * (f1a * f2a)
                o_v[slot][pl.ds(b + LANES, LANES)] = s1 * (f1b * f2b)

            pltpu.make_async_copy(
                o_v[slot], out_hbm.at[row, pl.ds(c * CHUNK, CHUNK)],
                sem_o[slot]).start()
        return carry

    lax.fori_loop(0, ROWS_PER_W, row_body, 0)

    pltpu.make_async_copy(i1s_v.at[pl.ds(0, CHUNK)], i1_v[0], sem_i).wait()
    pltpu.make_async_copy(i2s_v.at[pl.ds(0, CHUNK)], i2_v[0], sem_i).wait()
    for slot in range(2):
        pltpu.make_async_copy(
            o_v[slot], out_hbm.at[0, pl.ds(0, CHUNK)], sem_o[slot]).wait()


def kernel(facts, idx, weight):
    idx32 = idx.astype(jnp.int32)
    idx1 = idx32[:, 0]
    idx2 = idx32[:, 1]
    mesh = plsc.VectorSubcoreMesh(core_axis_name="c", subcore_axis_name="s")
    f = pl.kernel(
        _sc_body,
        out_type=jax.ShapeDtypeStruct((BATCH, NUM_RULES), jnp.float32),
        mesh=mesh,
        compiler_params=pltpu.CompilerParams(
            needs_layout_passes=False, use_tc_tiling_on_sc=True),
        scratch_types=[
            pltpu.VMEM((INPUT_DIM,), jnp.float32),
            pltpu.VMEM((NUM_RULES,), jnp.bfloat16),
            [pltpu.VMEM((CHUNK,), jnp.int32)] * 2,
            [pltpu.VMEM((CHUNK,), jnp.int32)] * 2,
            [pltpu.VMEM((CHUNK,), jnp.float32)] * 2,
            pltpu.VMEM_SHARED((NUM_RULES,), jnp.int32),
            pltpu.VMEM_SHARED((NUM_RULES,), jnp.int32),
            pltpu.SemaphoreType.DMA,
            [pltpu.SemaphoreType.DMA] * 2,
        ],
    )
    return f(facts, idx1, idx2, weight)


# speedup vs baseline: 6.2065x; 2.9408x over previous
import jax
import jax.numpy as jnp
from jax import lax
from jax.experimental import pallas as pl
from jax.experimental.pallas import tpu as pltpu
from jax.experimental.pallas import tpu_sc as plsc

BATCH = 1024
INPUT_DIM = 100000
NUM_RULES = 32768
LANES = 16
NUM_CORES = 2
NUM_SUBCORES = 16
NUM_WORKERS = NUM_CORES * NUM_SUBCORES
RULES_PER_W = NUM_RULES // NUM_WORKERS
KR = 16
NCHUNK = RULES_PER_W // KR
GROUPS = BATCH // LANES


def _sc_body(ft_hbm, idx1_hbm, idx2_hbm, w_hbm, out_hbm,
             i1_v, i2_v, ws_v, scale_v, f1_v, f2_v, o_v, sem_g, sem_o):
    wid = lax.axis_index("s") * NUM_CORES + lax.axis_index("c")
    base = wid * RULES_PER_W

    pltpu.sync_copy(idx1_hbm.at[pl.ds(base, RULES_PER_W)], i1_v)
    pltpu.sync_copy(idx2_hbm.at[pl.ds(base, RULES_PER_W)], i2_v)
    pltpu.sync_copy(w_hbm.at[pl.ds(base, RULES_PER_W)], ws_v)

    @plsc.parallel_loop(0, RULES_PER_W // LANES, unroll=4)
    def s_body(k):
        b = k * LANES
        w = ws_v[pl.ds(b, LANES)]
        scale_v[pl.ds(b, LANES)] = 1.0 / (1.0 + jnp.exp(-w))

    def start_gathers(c, slot):
        i1vec = i1_v[pl.ds(c * KR, KR)]
        i2vec = i2_v[pl.ds(c * KR, KR)]
        pltpu.make_async_copy(ft_hbm.at[i1vec], f1_v[slot], sem_g[slot]).start()
        pltpu.make_async_copy(ft_hbm.at[i2vec], f2_v[slot], sem_g[slot]).start()

    def wait_gathers(slot):
        zeros = i1_v[pl.ds(0, KR)]
        pltpu.make_async_copy(ft_hbm.at[zeros], f1_v[slot], sem_g[slot]).wait()
        pltpu.make_async_copy(ft_hbm.at[zeros], f2_v[slot], sem_g[slot]).wait()

    def wait_out(slot):
        pltpu.make_async_copy(
            o_v[slot], out_hbm.at[pl.ds(0, KR), :], sem_o[slot]).wait()

    start_gathers(0, 0)

    def pair_body(t, carry):
        for sub in range(2):
            c = t * 2 + sub
            slot = sub
            wait_gathers(slot)
            nxt = lax.rem(c + 1, NCHUNK)
            start_gathers(nxt, 1 - slot)

            @pl.when(c >= 2)
            def _():
                wait_out(slot)

            sv = scale_v[pl.ds(c * KR, KR)]
            for rl in range(KR):
                sbc = jax.lax.broadcast(sv[rl], (LANES,))

                @plsc.parallel_loop(0, GROUPS, unroll=4)
                def g_body(g, rl=rl, slot=slot, sbc=sbc):
                    b = g * LANES
                    v1 = f1_v[slot][rl, pl.ds(b, LANES)]
                    v2 = f2_v[slot][rl, pl.ds(b, LANES)]
                    o_v[slot][rl, pl.ds(b, LANES)] = sbc * (v1 * v2)
            pltpu.make_async_copy(
                o_v[slot], out_hbm.at[pl.ds(base + c * KR, KR), :],
                sem_o[slot]).start()
        return carry

    lax.fori_loop(0, NCHUNK // 2, pair_body, 0)

    wait_gathers(0)
    for slot in range(2):
        wait_out(slot)


def kernel(facts, idx, weight):
    idx32 = idx.astype(jnp.int32)
    idx1 = idx32[:, 0]
    idx2 = idx32[:, 1]
    ft = facts.T
    mesh = plsc.VectorSubcoreMesh(core_axis_name="c", subcore_axis_name="s")
    f = pl.kernel(
        _sc_body,
        out_type=jax.ShapeDtypeStruct((NUM_RULES, BATCH), jnp.float32),
        mesh=mesh,
        compiler_params=pltpu.CompilerParams(needs_layout_passes=False),
        scratch_types=[
            pltpu.VMEM((RULES_PER_W,), jnp.int32),
            pltpu.VMEM((RULES_PER_W,), jnp.int32),
            pltpu.VMEM((RULES_PER_W,), jnp.float32),
            pltpu.VMEM((RULES_PER_W,), jnp.float32),
            [pltpu.VMEM((KR, BATCH), jnp.float32)] * 2,
            [pltpu.VMEM((KR, BATCH), jnp.float32)] * 2,
            [pltpu.VMEM((KR, BATCH), jnp.float32)] * 2,
            [pltpu.SemaphoreType.DMA] * 2,
            [pltpu.SemaphoreType.DMA] * 2,
        ],
    )
    out_t = f(ft, idx1, idx2, weight)
    return out_t.T
